# Initial kernel scaffold; baseline (speedup 1.0000x reference)
import functools

import jax
import jax.numpy as jnp
from jax import lax
from jax.experimental import pallas as pl
from jax.experimental.pallas import tpu as pltpu
from jax.experimental.pallas import tpu_sc as plsc

N = 10000
E = 320000
NHID = 128
NLAYER = 3
NGRAPH = 128
NCLASS = 10

CHUNK = 128
NCHUNKS = E // CHUNK
NCORES = 2
NSUB = 16
NWORKERS = NCORES * NSUB
ROWS_PER_TILE = N // NSUB

BR = 1000
NBLK = N // BR


def _sc_aggregate(h, src, dst):
    mesh = plsc.VectorSubcoreMesh(core_axis_name="c", subcore_axis_name="s")

    @functools.partial(
        pl.kernel,
        mesh=mesh,
        out_type=jax.ShapeDtypeStruct((NCORES, N, NHID), jnp.float32),
        scratch_types=[
            pltpu.VMEM((CHUNK,), jnp.int32),
            pltpu.VMEM((CHUNK,), jnp.int32),
            pltpu.VMEM((CHUNK, NHID), jnp.float32),
            pltpu.VMEM_SHARED((N, NHID), jnp.float32),
            pltpu.SemaphoreType.DMA,
        ],
    )
    def agg_kernel(h_hbm, src_hbm, dst_hbm, out_hbm, src_v, dst_v, rows_v, acc, sem):
        c = lax.axis_index("c")
        s = lax.axis_index("s")
        w = s * NCORES + c

        row0 = s * ROWS_PER_TILE
        pltpu.sync_copy(h_hbm.at[pl.ds(row0, ROWS_PER_TILE)],
                        acc.at[pl.ds(row0, ROWS_PER_TILE)])
        plsc.subcore_barrier()

        start = w * NCHUNKS // NWORKERS
        end = (w + 1) * NCHUNKS // NWORKERS

        def body(i, carry):
            base = i * CHUNK
            pltpu.sync_copy(src_hbm.at[pl.ds(base, CHUNK)], src_v)
            pltpu.sync_copy(dst_hbm.at[pl.ds(base, CHUNK)], dst_v)
            pltpu.async_copy(h_hbm.at[src_v], rows_v, sem).wait()
            pltpu.sync_copy(rows_v, acc.at[dst_v], add=True)
            return carry

        lax.fori_loop(start, end, body, 0)
        plsc.subcore_barrier()

        pltpu.sync_copy(acc.at[pl.ds(row0, ROWS_PER_TILE)],
                        out_hbm.at[c, pl.ds(row0, ROWS_PER_TILE)])

    return agg_kernel(h, src, dst)


def _mlp_body(p_ref, h_ref, w1_ref, b1_ref, w2_ref, b2_ref, o_ref):
    z = p_ref[0] + p_ref[1] - h_ref[...]
    z = jnp.dot(z, w1_ref[...], preferred_element_type=jnp.float32) + b1_ref[...]
    z = jnp.maximum(z, 0.0)
    z = jnp.dot(z, w2_ref[...], preferred_element_type=jnp.float32) + b2_ref[...]
    o_ref[...] = jnp.maximum(z, 0.0)


def _tc_mlp(p, h, w1, b1, w2, b2):
    return pl.pallas_call(
        _mlp_body,
        grid=(NBLK,),
        in_specs=[
            pl.BlockSpec((NCORES, BR, NHID), lambda i: (0, i, 0)),
            pl.BlockSpec((BR, NHID), lambda i: (i, 0)),
            pl.BlockSpec((NHID, NHID), lambda i: (0, 0)),
            pl.BlockSpec((1, NHID), lambda i: (0, 0)),
            pl.BlockSpec((NHID, NHID), lambda i: (0, 0)),
            pl.BlockSpec((1, NHID), lambda i: (0, 0)),
        ],
        out_specs=pl.BlockSpec((BR, NHID), lambda i: (i, 0)),
        out_shape=jax.ShapeDtypeStruct((N, NHID), jnp.float32),
    )(p, h, w1, b1, w2, b2)


def _pool_head_body(h_ref, b_ref, pw_ref, pb_ref, rw_ref, rb_ref, o_ref, g_acc):
    i = pl.program_id(0)

    @pl.when(i == 0)
    def _init():
        g_acc[...] = jnp.zeros_like(g_acc)

    bvals = b_ref[0]
    mask = (lax.broadcasted_iota(jnp.int32, (NGRAPH, BR), 0) == bvals
            ).astype(jnp.float32)
    g_acc[...] += jnp.dot(mask, h_ref[...], preferred_element_type=jnp.float32)

    @pl.when(i == NBLK - 1)
    def _head():
        g = g_acc[...]
        z = jnp.dot(g, pw_ref[...], preferred_element_type=jnp.float32) + pb_ref[...]
        z = jnp.maximum(z, 0.0)
        logits = jnp.dot(z, rw_ref[...], preferred_element_type=jnp.float32) + rb_ref[...]
        col = lax.broadcasted_iota(jnp.int32, (NGRAPH, NHID), 1)
        logits = jnp.where(col < NCLASS, logits, jnp.float32(-1e30))
        m = jnp.max(logits, axis=1, keepdims=True)
        lse = m + jnp.log(jnp.sum(jnp.exp(logits - m), axis=1, keepdims=True))
        o_ref[...] = logits - lse


def _tc_pool_head(h, batch3, post_w, post_b, rw_pad, rb_pad):
    return pl.pallas_call(
        _pool_head_body,
        grid=(NBLK,),
        in_specs=[
            pl.BlockSpec((BR, NHID), lambda i: (i, 0)),
            pl.BlockSpec((1, 1, BR), lambda i: (i, 0, 0)),
            pl.BlockSpec((NHID, NHID), lambda i: (0, 0)),
            pl.BlockSpec((1, NHID), lambda i: (0, 0)),
            pl.BlockSpec((NHID, NHID), lambda i: (0, 0)),
            pl.BlockSpec((1, NHID), lambda i: (0, 0)),
        ],
        out_specs=pl.BlockSpec((NGRAPH, NHID), lambda i: (0, 0)),
        out_shape=jax.ShapeDtypeStruct((NGRAPH, NHID), jnp.float32),
        scratch_shapes=[pltpu.VMEM((NGRAPH, NHID), jnp.float32)],
    )(h, batch3, post_w, post_b, rw_pad, rb_pad)


def kernel(x, edge_index, batch, conv_w1, conv_b1, conv_w2, conv_b2,
           post_w, post_b, read_w, read_b):
    src = edge_index[0]
    dst = edge_index[1]
    h = x
    for l in range(NLAYER):
        p = _sc_aggregate(h, src, dst)
        h = _tc_mlp(p, h, conv_w1[l], conv_b1[l].reshape(1, NHID),
                    conv_w2[l], conv_b2[l].reshape(1, NHID))

    batch3 = batch.reshape(NBLK, 1, BR)
    rw_pad = jnp.zeros((NHID, NHID), jnp.float32).at[:, :NCLASS].set(read_w)
    rb_pad = jnp.zeros((1, NHID), jnp.float32).at[0, :NCLASS].set(read_b)
    out = _tc_pool_head(h, batch3, post_w, post_b.reshape(1, NHID),
                        rw_pad, rb_pad)
    return out[:, :NCLASS]


# speedup vs baseline: 5.6716x; 5.6716x over previous
import functools

import jax
import jax.numpy as jnp
from jax import lax
from jax.experimental import pallas as pl
from jax.experimental.pallas import tpu as pltpu
from jax.experimental.pallas import tpu_sc as plsc

N = 10000
E = 320000
NHID = 128
NLAYER = 3
NGRAPH = 128
NCLASS = 10

CHUNK = 128
NCHUNKS = E // CHUNK
NCORES = 2
NSUB = 16
NWORKERS = NCORES * NSUB
ROWS_A = 624
ROWS_LAST = N - (NSUB - 1) * ROWS_A

BR = 1000
NBLK = N // BR


def _sc_aggregate(h, src, dst):
    mesh = plsc.VectorSubcoreMesh(core_axis_name="c", subcore_axis_name="s")

    @functools.partial(
        pl.kernel,
        mesh=mesh,
        out_type=jax.ShapeDtypeStruct((NCORES, N, NHID), jnp.float32),
        scratch_types=[
            pltpu.VMEM((CHUNK,), jnp.int32),
            pltpu.VMEM((CHUNK,), jnp.int32),
            pltpu.VMEM((CHUNK, NHID), jnp.float32),
            pltpu.VMEM_SHARED((N, NHID), jnp.float32),
            pltpu.SemaphoreType.DMA,
        ],
    )
    def agg_kernel(h_hbm, src_hbm, dst_hbm, out_hbm, src_v, dst_v, rows_v, acc, sem):
        c = lax.axis_index("c")
        s = lax.axis_index("s")
        w = s * NCORES + c

        row0 = pl.multiple_of(s * ROWS_A, 8)

        @pl.when(s < NSUB - 1)
        def _init_a():
            pltpu.sync_copy(h_hbm.at[pl.ds(row0, ROWS_A)],
                            acc.at[pl.ds(row0, ROWS_A)])

        @pl.when(s == NSUB - 1)
        def _init_b():
            pltpu.sync_copy(h_hbm.at[pl.ds((NSUB - 1) * ROWS_A, ROWS_LAST)],
                            acc.at[pl.ds((NSUB - 1) * ROWS_A, ROWS_LAST)])

        plsc.subcore_barrier()

        start = w * NCHUNKS // NWORKERS
        end = (w + 1) * NCHUNKS // NWORKERS

        def body(i, carry):
            base = i * CHUNK
            pltpu.sync_copy(src_hbm.at[pl.ds(base, CHUNK)], src_v)
            pltpu.sync_copy(dst_hbm.at[pl.ds(base, CHUNK)], dst_v)
            pltpu.async_copy(h_hbm.at[src_v], rows_v, sem).wait()
            pltpu.sync_copy(rows_v, acc.at[dst_v], add=True)
            return carry

        lax.fori_loop(start, end, body, 0)
        plsc.subcore_barrier()

        @pl.when(s < NSUB - 1)
        def _out_a():
            pltpu.sync_copy(acc.at[pl.ds(row0, ROWS_A)],
                            out_hbm.at[c, pl.ds(row0, ROWS_A)])

        @pl.when(s == NSUB - 1)
        def _out_b():
            pltpu.sync_copy(acc.at[pl.ds((NSUB - 1) * ROWS_A, ROWS_LAST)],
                            out_hbm.at[c, pl.ds((NSUB - 1) * ROWS_A, ROWS_LAST)])

    return agg_kernel(h, src, dst)


def _mlp_body(p_ref, h_ref, w1_ref, b1_ref, w2_ref, b2_ref, o_ref):
    z = p_ref[0] + p_ref[1] - h_ref[...]
    z = jnp.dot(z, w1_ref[...], preferred_element_type=jnp.float32) + b1_ref[...]
    z = jnp.maximum(z, 0.0)
    z = jnp.dot(z, w2_ref[...], preferred_element_type=jnp.float32) + b2_ref[...]
    o_ref[...] = jnp.maximum(z, 0.0)


def _tc_mlp(p, h, w1, b1, w2, b2):
    return pl.pallas_call(
        _mlp_body,
        grid=(NBLK,),
        in_specs=[
            pl.BlockSpec((NCORES, BR, NHID), lambda i: (0, i, 0)),
            pl.BlockSpec((BR, NHID), lambda i: (i, 0)),
            pl.BlockSpec((NHID, NHID), lambda i: (0, 0)),
            pl.BlockSpec((1, NHID), lambda i: (0, 0)),
            pl.BlockSpec((NHID, NHID), lambda i: (0, 0)),
            pl.BlockSpec((1, NHID), lambda i: (0, 0)),
        ],
        out_specs=pl.BlockSpec((BR, NHID), lambda i: (i, 0)),
        out_shape=jax.ShapeDtypeStruct((N, NHID), jnp.float32),
    )(p, h, w1, b1, w2, b2)


def _pool_head_body(h_ref, b_ref, pw_ref, pb_ref, rw_ref, rb_ref, o_ref, g_acc):
    i = pl.program_id(0)

    @pl.when(i == 0)
    def _init():
        g_acc[...] = jnp.zeros_like(g_acc)

    bvals = b_ref[0]
    mask = (lax.broadcasted_iota(jnp.int32, (NGRAPH, BR), 0) == bvals
            ).astype(jnp.float32)
    g_acc[...] += jnp.dot(mask, h_ref[...], preferred_element_type=jnp.float32)

    @pl.when(i == NBLK - 1)
    def _head():
        g = g_acc[...]
        z = jnp.dot(g, pw_ref[...], preferred_element_type=jnp.float32) + pb_ref[...]
        z = jnp.maximum(z, 0.0)
        logits = jnp.dot(z, rw_ref[...], preferred_element_type=jnp.float32) + rb_ref[...]
        col = lax.broadcasted_iota(jnp.int32, (NGRAPH, NHID), 1)
        logits = jnp.where(col < NCLASS, logits, jnp.float32(-1e30))
        m = jnp.max(logits, axis=1, keepdims=True)
        lse = m + jnp.log(jnp.sum(jnp.exp(logits - m), axis=1, keepdims=True))
        o_ref[...] = logits - lse


def _tc_pool_head(h, batch3, post_w, post_b, rw_pad, rb_pad):
    return pl.pallas_call(
        _pool_head_body,
        grid=(NBLK,),
        in_specs=[
            pl.BlockSpec((BR, NHID), lambda i: (i, 0)),
            pl.BlockSpec((1, 1, BR), lambda i: (i, 0, 0)),
            pl.BlockSpec((NHID, NHID), lambda i: (0, 0)),
            pl.BlockSpec((1, NHID), lambda i: (0, 0)),
            pl.BlockSpec((NHID, NHID), lambda i: (0, 0)),
            pl.BlockSpec((1, NHID), lambda i: (0, 0)),
        ],
        out_specs=pl.BlockSpec((NGRAPH, NHID), lambda i: (0, 0)),
        out_shape=jax.ShapeDtypeStruct((NGRAPH, NHID), jnp.float32),
        scratch_shapes=[pltpu.VMEM((NGRAPH, NHID), jnp.float32)],
    )(h, batch3, post_w, post_b, rw_pad, rb_pad)


def kernel(x, edge_index, batch, conv_w1, conv_b1, conv_w2, conv_b2,
           post_w, post_b, read_w, read_b):
    src = edge_index[0]
    dst = edge_index[1]
    h = x
    for l in range(NLAYER):
        p = _sc_aggregate(h, src, dst)
        h = _tc_mlp(p, h, conv_w1[l], conv_b1[l].reshape(1, NHID),
                    conv_w2[l], conv_b2[l].reshape(1, NHID))

    batch3 = batch.reshape(NBLK, 1, BR)
    rw_pad = jnp.zeros((NHID, NHID), jnp.float32).at[:, :NCLASS].set(read_w)
    rb_pad = jnp.zeros((1, NHID), jnp.float32).at[0, :NCLASS].set(read_b)
    out = _tc_pool_head(h, batch3, post_w, post_b.reshape(1, NHID),
                        rw_pad, rb_pad)
    return out[:, :NCLASS]
